# Initial kernel scaffold; baseline (speedup 1.0000x reference)
import jax
import jax.numpy as jnp
from jax.experimental import pallas as pl


def kernel(x, edge_index, W_self1, W_neigh1, b1, W_self2, W_neigh2, b2):
    raise NotImplementedError("write your pallas kernel here")


# speedup vs baseline: 6.5730x; 6.5730x over previous
import functools

import jax
import jax.numpy as jnp
from jax import lax
from jax.experimental import pallas as pl
from jax.experimental.pallas import tpu as pltpu
from jax.experimental.pallas import tpu_sc as plsc

N = 10000
E = 320000
D_IN = 128
D_H = 128
D_OUT = 64

NP = 10240
NC = 2
NS = 16
NW = NC * NS
K = 128
NCHUNKS = E // K
RPT = NP // NS


def _make_agg(D, with_deg):
    mesh = plsc.VectorSubcoreMesh(core_axis_name="c", subcore_axis_name="s")
    out_type = [jax.ShapeDtypeStruct((NC, NP, D), jnp.float32)]
    if with_deg:
        out_type.append(jax.ShapeDtypeStruct((NC, NP), jnp.float32))

    scratch = [
        pltpu.VMEM((K,), jnp.int32),
        pltpu.VMEM((K,), jnp.int32),
        pltpu.VMEM((K, D), jnp.float32),
        pltpu.VMEM((K,), jnp.float32),
        pltpu.VMEM_SHARED((NP, D), jnp.float32),
        pltpu.VMEM_SHARED((NP,), jnp.float32),
        pltpu.SemaphoreType.DMA,
    ]

    @functools.partial(
        pl.kernel,
        out_type=tuple(out_type),
        mesh=mesh,
        scratch_types=scratch,
        compiler_params=pltpu.CompilerParams(use_tc_tiling_on_sc=False),
    )
    def agg(table, srcs, dsts, zrows, zvec, *rest):
        if with_deg:
            acc_out, deg_out = rest[0], rest[1]
            rest = rest[2:]
        else:
            acc_out = rest[0]
            rest = rest[1:]
        src_v, dst_v, rows_v, ones_v, acc_sh, deg_sh, sem = rest

        cid = lax.axis_index("c")
        sid = lax.axis_index("s")
        w = sid * NC + cid

        lo = sid * RPT
        pltpu.sync_copy(zrows.at[pl.ds(lo, RPT)], acc_sh.at[pl.ds(lo, RPT)])
        if with_deg:
            pltpu.sync_copy(zvec.at[pl.ds(lo, RPT)], deg_sh.at[pl.ds(lo, RPT)])
            for j in range(K // 16):
                ones_v[pl.ds(j * 16, 16)] = jnp.ones((16,), jnp.float32)
        plsc.subcore_barrier()

        n_w = (NCHUNKS - 1 - w) // NW + 1

        def body(i, carry):
            c = w + i * NW
            off = pl.multiple_of(c * K, K)
            pltpu.sync_copy(srcs.at[pl.ds(off, K)], src_v)
            pltpu.sync_copy(dsts.at[pl.ds(off, K)], dst_v)
            pltpu.async_copy(table.at[src_v], rows_v, sem).wait()
            pltpu.sync_copy(rows_v, acc_sh.at[dst_v], add=True)
            if with_deg:
                pltpu.sync_copy(ones_v, deg_sh.at[dst_v], add=True)
            return carry

        lax.fori_loop(0, n_w, body, 0)

        plsc.subcore_barrier()
        pltpu.sync_copy(acc_sh.at[pl.ds(lo, RPT)], acc_out.at[cid, pl.ds(lo, RPT)])
        if with_deg:
            pltpu.sync_copy(deg_sh.at[pl.ds(lo, RPT)], deg_out.at[cid, pl.ds(lo, RPT)])

    return agg


_agg128 = _make_agg(D_H, True)
_agg64 = _make_agg(D_OUT, False)

BR = 512


def _dense1_body(x_ref, sx_ref, deg_ref, ws1_ref, wn1_ref, b1_ref,
                 ws2_ref, wn2_ref, b2_ref, p_ref, q_ref):
    s = sx_ref[0] + sx_ref[1]
    d = deg_ref[0] + deg_ref[1]
    recip = 1.0 / jnp.maximum(d, 1.0)
    hn = s * recip[:, None]
    xb = x_ref[...]
    h1 = (jnp.dot(xb, ws1_ref[...], preferred_element_type=jnp.float32)
          + jnp.dot(hn, wn1_ref[...], preferred_element_type=jnp.float32)
          + b1_ref[...])
    g = jnp.maximum(h1, 0.0)
    p_ref[...] = jnp.dot(g, wn2_ref[...], preferred_element_type=jnp.float32)
    q_ref[...] = (jnp.dot(g, ws2_ref[...], preferred_element_type=jnp.float32)
                  + b2_ref[...])


def _dense1(xp, sx, deg, ws1, wn1, b1, ws2, wn2, b2):
    grid = (NP // BR,)
    return pl.pallas_call(
        _dense1_body,
        grid=grid,
        in_specs=[
            pl.BlockSpec((BR, D_IN), lambda i: (i, 0)),
            pl.BlockSpec((NC, BR, D_H), lambda i: (0, i, 0)),
            pl.BlockSpec((NC, BR), lambda i: (0, i)),
            pl.BlockSpec((D_IN, D_H), lambda i: (0, 0)),
            pl.BlockSpec((D_IN, D_H), lambda i: (0, 0)),
            pl.BlockSpec((1, D_H), lambda i: (0, 0)),
            pl.BlockSpec((D_H, D_OUT), lambda i: (0, 0)),
            pl.BlockSpec((D_H, D_OUT), lambda i: (0, 0)),
            pl.BlockSpec((1, D_OUT), lambda i: (0, 0)),
        ],
        out_specs=[
            pl.BlockSpec((BR, D_OUT), lambda i: (i, 0)),
            pl.BlockSpec((BR, D_OUT), lambda i: (i, 0)),
        ],
        out_shape=[
            jax.ShapeDtypeStruct((NP, D_OUT), jnp.float32),
            jax.ShapeDtypeStruct((NP, D_OUT), jnp.float32),
        ],
    )(xp, sx, deg, ws1, wn1, b1, ws2, wn2, b2)


def _dense2_body(q_ref, sp_ref, deg_ref, out_ref):
    s = sp_ref[0] + sp_ref[1]
    d = deg_ref[0] + deg_ref[1]
    recip = 1.0 / jnp.maximum(d, 1.0)
    out_ref[...] = q_ref[...] + s * recip[:, None]


def _dense2(q, sp, deg):
    grid = (NP // BR,)
    return pl.pallas_call(
        _dense2_body,
        grid=grid,
        in_specs=[
            pl.BlockSpec((BR, D_OUT), lambda i: (i, 0)),
            pl.BlockSpec((NC, BR, D_OUT), lambda i: (0, i, 0)),
            pl.BlockSpec((NC, BR), lambda i: (0, i)),
        ],
        out_specs=pl.BlockSpec((BR, D_OUT), lambda i: (i, 0)),
        out_shape=jax.ShapeDtypeStruct((NP, D_OUT), jnp.float32),
    )(q, sp, deg)


@jax.jit
def kernel(x, edge_index, W_self1, W_neigh1, b1, W_self2, W_neigh2, b2):
    src = edge_index[0]
    dst = edge_index[1]

    z128 = jnp.zeros((NP, D_H), jnp.float32)
    z64 = jnp.zeros((NP, D_OUT), jnp.float32)
    zv = jnp.zeros((NP,), jnp.float32)

    sx, deg = _agg128(x, src, dst, z128, zv)

    xp = jnp.pad(x, ((0, NP - N), (0, 0)))
    p, q = _dense1(xp, sx, deg,
                   W_self1, W_neigh1, b1.reshape(1, D_H),
                   W_self2, W_neigh2, b2.reshape(1, D_OUT))

    (sp,) = _agg64(p, src, dst, z64, zv)

    out = _dense2(q, sp, deg)
    return out[:N]


# speedup vs baseline: 12.5350x; 1.9070x over previous
import functools

import jax
import jax.numpy as jnp
from jax import lax
from jax.experimental import pallas as pl
from jax.experimental.pallas import tpu as pltpu
from jax.experimental.pallas import tpu_sc as plsc

N = 10000
E = 320000
D_IN = 128
D_H = 128
D_OUT = 64

NP = 10240
NC = 2
NS = 16
NW = NC * NS
K = 128
NCHUNKS = E // K
RPT = NP // NS


def _make_agg(D, with_deg):
    mesh = plsc.VectorSubcoreMesh(core_axis_name="c", subcore_axis_name="s")
    out_type = [jax.ShapeDtypeStruct((NC, NP, D), jnp.float32)]
    if with_deg:
        out_type.append(jax.ShapeDtypeStruct((NC, NP), jnp.float32))

    scratch = [
        pltpu.VMEM((K,), jnp.int32),
        pltpu.VMEM((K,), jnp.int32),
        pltpu.VMEM((K, D), jnp.float32),
        pltpu.VMEM((K,), jnp.int32),
        pltpu.VMEM((K,), jnp.int32),
        pltpu.VMEM((K, D), jnp.float32),
        pltpu.VMEM((K,), jnp.float32),
        pltpu.VMEM_SHARED((NP, D), jnp.float32),
        pltpu.VMEM_SHARED((NP,), jnp.float32),
        pltpu.SemaphoreType.DMA,
        pltpu.SemaphoreType.DMA,
        pltpu.SemaphoreType.DMA,
        pltpu.SemaphoreType.DMA,
        pltpu.SemaphoreType.DMA,
        pltpu.SemaphoreType.DMA,
    ]

    @functools.partial(
        pl.kernel,
        out_type=tuple(out_type),
        mesh=mesh,
        scratch_types=scratch,
        compiler_params=pltpu.CompilerParams(use_tc_tiling_on_sc=False),
    )
    def agg(table, srcs, dsts, zrows, zvec, *rest):
        if with_deg:
            acc_out, deg_out = rest[0], rest[1]
            rest = rest[2:]
        else:
            acc_out = rest[0]
            rest = rest[1:]
        (src0, dst0, rows0, src1, dst1, rows1, ones_v, acc_sh, deg_sh,
         ssem0, dsem0, ssem1, dsem1, gsem0, gsem1) = rest

        cid = lax.axis_index("c")
        sid = lax.axis_index("s")
        w = sid * NC + cid

        lo = sid * RPT
        pltpu.sync_copy(zrows.at[pl.ds(lo, RPT)], acc_sh.at[pl.ds(lo, RPT)])
        if with_deg:
            pltpu.sync_copy(zvec.at[pl.ds(lo, RPT)], deg_sh.at[pl.ds(lo, RPT)])
            for j in range(K // 16):
                ones_v[pl.ds(j * 16, 16)] = jnp.ones((16,), jnp.float32)
        plsc.subcore_barrier()

        n_w = (NCHUNKS - 1 - w) // NW + 1

        def off(j):
            return (w + j * NW) * K

        def scat(rows_v, dst_v):
            pltpu.sync_copy(rows_v, acc_sh.at[dst_v], add=True)
            if with_deg:
                pltpu.sync_copy(ones_v, deg_sh.at[dst_v], add=True)

        pltpu.sync_copy(srcs.at[pl.ds(off(0), K)], src0)
        pltpu.sync_copy(dsts.at[pl.ds(off(0), K)], dst0)
        pltpu.async_copy(table.at[src0], rows0, gsem0)
        pltpu.async_copy(srcs.at[pl.ds(off(1), K)], src1, ssem1)
        pltpu.async_copy(dsts.at[pl.ds(off(1), K)], dst1, dsem1)

        n_pairs = (n_w + 1) // 2

        def body(g, carry):
            j1 = 2 * g + 1
            c2 = 2 * g + 2
            c3 = 2 * g + 3

            @pl.when(j1 < n_w)
            def _():
                pltpu.make_async_copy(srcs.at[pl.ds(0, K)], src1, ssem1).wait()
                pltpu.async_copy(table.at[src1], rows1, gsem1)

            pltpu.make_async_copy(table.at[src0], rows0, gsem0).wait()

            @pl.when(c2 < n_w)
            def _():
                pltpu.async_copy(srcs.at[pl.ds(off(c2), K)], src0, ssem0)

            @pl.when(g > 0)
            def _():
                pltpu.make_async_copy(dsts.at[pl.ds(0, K)], dst0, dsem0).wait()

            scat(rows0, dst0)

            @pl.when(c2 < n_w)
            def _():
                pltpu.async_copy(dsts.at[pl.ds(off(c2), K)], dst0, dsem0)

            @pl.when(j1 < n_w)
            def _():
                pltpu.make_async_copy(table.at[src1], rows1, gsem1).wait()

                @pl.when(c2 < n_w)
                def _():
                    pltpu.make_async_copy(srcs.at[pl.ds(0, K)], src0, ssem0).wait()
                    pltpu.async_copy(table.at[src0], rows0, gsem0)

                @pl.when(c3 < n_w)
                def _():
                    pltpu.async_copy(srcs.at[pl.ds(off(c3), K)], src1, ssem1)

                pltpu.make_async_copy(dsts.at[pl.ds(0, K)], dst1, dsem1).wait()
                scat(rows1, dst1)

                @pl.when(c3 < n_w)
                def _():
                    pltpu.async_copy(dsts.at[pl.ds(off(c3), K)], dst1, dsem1)

            return carry

        lax.fori_loop(0, n_pairs, body, 0)

        plsc.subcore_barrier()
        pltpu.sync_copy(acc_sh.at[pl.ds(lo, RPT)], acc_out.at[cid, pl.ds(lo, RPT)])
        if with_deg:
            pltpu.sync_copy(deg_sh.at[pl.ds(lo, RPT)], deg_out.at[cid, pl.ds(lo, RPT)])

    return agg


_agg128 = _make_agg(D_H, True)
_agg64 = _make_agg(D_OUT, False)

BR = 512


def _dense1_body(x_ref, sx_ref, deg_ref, ws1_ref, wn1_ref, b1_ref,
                 ws2_ref, wn2_ref, b2_ref, p_ref, q_ref):
    s = sx_ref[0] + sx_ref[1]
    d = deg_ref[0] + deg_ref[1]
    recip = 1.0 / jnp.maximum(d, 1.0)
    hn = s * recip[:, None]
    xb = x_ref[...]
    h1 = (jnp.dot(xb, ws1_ref[...], preferred_element_type=jnp.float32)
          + jnp.dot(hn, wn1_ref[...], preferred_element_type=jnp.float32)
          + b1_ref[...])
    g = jnp.maximum(h1, 0.0)
    p_ref[...] = jnp.dot(g, wn2_ref[...], preferred_element_type=jnp.float32)
    q_ref[...] = (jnp.dot(g, ws2_ref[...], preferred_element_type=jnp.float32)
                  + b2_ref[...])


def _dense1(xp, sx, deg, ws1, wn1, b1, ws2, wn2, b2):
    grid = (NP // BR,)
    return pl.pallas_call(
        _dense1_body,
        grid=grid,
        in_specs=[
            pl.BlockSpec((BR, D_IN), lambda i: (i, 0)),
            pl.BlockSpec((NC, BR, D_H), lambda i: (0, i, 0)),
            pl.BlockSpec((NC, BR), lambda i: (0, i)),
            pl.BlockSpec((D_IN, D_H), lambda i: (0, 0)),
            pl.BlockSpec((D_IN, D_H), lambda i: (0, 0)),
            pl.BlockSpec((1, D_H), lambda i: (0, 0)),
            pl.BlockSpec((D_H, D_OUT), lambda i: (0, 0)),
            pl.BlockSpec((D_H, D_OUT), lambda i: (0, 0)),
            pl.BlockSpec((1, D_OUT), lambda i: (0, 0)),
        ],
        out_specs=[
            pl.BlockSpec((BR, D_OUT), lambda i: (i, 0)),
            pl.BlockSpec((BR, D_OUT), lambda i: (i, 0)),
        ],
        out_shape=[
            jax.ShapeDtypeStruct((NP, D_OUT), jnp.float32),
            jax.ShapeDtypeStruct((NP, D_OUT), jnp.float32),
        ],
    )(xp, sx, deg, ws1, wn1, b1, ws2, wn2, b2)


def _dense2_body(q_ref, sp_ref, deg_ref, out_ref):
    s = sp_ref[0] + sp_ref[1]
    d = deg_ref[0] + deg_ref[1]
    recip = 1.0 / jnp.maximum(d, 1.0)
    out_ref[...] = q_ref[...] + s * recip[:, None]


def _dense2(q, sp, deg):
    grid = (NP // BR,)
    return pl.pallas_call(
        _dense2_body,
        grid=grid,
        in_specs=[
            pl.BlockSpec((BR, D_OUT), lambda i: (i, 0)),
            pl.BlockSpec((NC, BR, D_OUT), lambda i: (0, i, 0)),
            pl.BlockSpec((NC, BR), lambda i: (0, i)),
        ],
        out_specs=pl.BlockSpec((BR, D_OUT), lambda i: (i, 0)),
        out_shape=jax.ShapeDtypeStruct((NP, D_OUT), jnp.float32),
    )(q, sp, deg)


@jax.jit
def kernel(x, edge_index, W_self1, W_neigh1, b1, W_self2, W_neigh2, b2):
    src = edge_index[0]
    dst = edge_index[1]

    z128 = jnp.zeros((NP, D_H), jnp.float32)
    z64 = jnp.zeros((NP, D_OUT), jnp.float32)
    zv = jnp.zeros((NP,), jnp.float32)

    sx, deg = _agg128(x, src, dst, z128, zv)

    xp = jnp.pad(x, ((0, NP - N), (0, 0)))
    p, q = _dense1(xp, sx, deg,
                   W_self1, W_neigh1, b1.reshape(1, D_H),
                   W_self2, W_neigh2, b2.reshape(1, D_OUT))

    (sp,) = _agg64(p, src, dst, z64, zv)

    out = _dense2(q, sp, deg)
    return out[:N]
